# baseline (speedup 1.0000x reference)
import functools

import jax
import jax.numpy as jnp
from jax import lax
from jax.experimental import pallas as pl
from jax.experimental.pallas import tpu as pltpu
from jax.experimental.pallas import tpu_sc as plsc

_N = 10000
_E = 320000
_D = 128
_H = 8
_DH = 16
_HOP = 5
_ALPHA = 0.15
_DFF = 512

_NT = 16
_BS = 64
_NB = 314
_EPT = _NB * _BS
_EPAD = _NT * _EPT
_NB15 = 290
_NPT = _N // _NT
_UCHUNKS = tuple((q * 32, 32) for q in range(19)) + ((608, 17),)
_HHD = 64

_f32 = jnp.float32


def _proj_body(x_ref, g_ref, b_ref, w_ref, ahf_ref, atf_ref, mh_ref,
               h_ref, feat_ref, eh_ref, et_ref):
    x = x_ref[...]
    m = jnp.mean(x, axis=1, keepdims=True)
    xc = x - m
    var = jnp.mean(xc * xc, axis=1, keepdims=True)
    h = xc * lax.rsqrt(var + 1e-5) * g_ref[...] + b_ref[...]
    feat = lax.dot_general(h, w_ref[...], (((1,), (1,)), ((), ())),
                           preferred_element_type=_f32)
    eh = lax.dot_general(feat * ahf_ref[...], mh_ref[...],
                         (((1,), (0,)), ((), ())), preferred_element_type=_f32)
    et = lax.dot_general(feat * atf_ref[...], mh_ref[...],
                         (((1,), (0,)), ((), ())), preferred_element_type=_f32)
    h_ref[...] = h
    feat_ref[...] = feat
    eh_ref[...] = eh
    et_ref[...] = et


_PROJ_NB = 400


def _proj(x, g, b, w, ahf, atf, mh):
    grid = _N // _PROJ_NB
    full = lambda i: (0, 0)
    blk = lambda i: (i, 0)
    return pl.pallas_call(
        _proj_body,
        grid=(grid,),
        in_specs=[
            pl.BlockSpec((_PROJ_NB, _D), blk),
            pl.BlockSpec((1, _D), full),
            pl.BlockSpec((1, _D), full),
            pl.BlockSpec((_D, _D), full),
            pl.BlockSpec((1, _D), full),
            pl.BlockSpec((1, _D), full),
            pl.BlockSpec((_D, _H), full),
        ],
        out_specs=[
            pl.BlockSpec((_PROJ_NB, _D), blk),
            pl.BlockSpec((_PROJ_NB, _D), blk),
            pl.BlockSpec((_PROJ_NB, _H), blk),
            pl.BlockSpec((_PROJ_NB, _H), blk),
        ],
        out_shape=[
            jax.ShapeDtypeStruct((_N, _D), _f32),
            jax.ShapeDtypeStruct((_N, _D), _f32),
            jax.ShapeDtypeStruct((_N, _H), _f32),
            jax.ShapeDtypeStruct((_N, _H), _f32),
        ],
    )(x, g, b, w, ahf, atf, mh)


def _ffn_body(f_ref, h_ref, g_ref, b_ref, w1_ref, b1_ref, w2_ref, b2_ref,
              o_ref):
    rst = f_ref[...] + h_ref[...]
    m = jnp.mean(rst, axis=1, keepdims=True)
    xc = rst - m
    var = jnp.mean(xc * xc, axis=1, keepdims=True)
    h2 = xc * lax.rsqrt(var + 1e-5) * g_ref[...] + b_ref[...]
    t = lax.dot_general(h2, w1_ref[...], (((1,), (1,)), ((), ())),
                        preferred_element_type=_f32) + b1_ref[...]
    t = jnp.maximum(t, 0.0)
    o_ref[...] = lax.dot_general(t, w2_ref[...], (((1,), (1,)), ((), ())),
                                 preferred_element_type=_f32) + b2_ref[...] + rst


def _ffn(f, h, g, b, w1, b1, w2, b2):
    grid = _N // _PROJ_NB
    full = lambda i: (0, 0)
    blk = lambda i: (i, 0)
    return pl.pallas_call(
        _ffn_body,
        grid=(grid,),
        in_specs=[
            pl.BlockSpec((_PROJ_NB, _D), blk),
            pl.BlockSpec((_PROJ_NB, _D), blk),
            pl.BlockSpec((1, _D), full),
            pl.BlockSpec((1, _D), full),
            pl.BlockSpec((_DFF, _D), full),
            pl.BlockSpec((1, _DFF), full),
            pl.BlockSpec((_D, _DFF), full),
            pl.BlockSpec((1, _D), full),
        ],
        out_specs=pl.BlockSpec((_PROJ_NB, _D), blk),
        out_shape=jax.ShapeDtypeStruct((_N, _D), _f32),
    )(f, h, g, b, w1, b1, w2, b2)


_MESH = plsc.VectorSubcoreMesh(core_axis_name="c", subcore_axis_name="s")


@functools.partial(
    pl.kernel,
    out_type=(
        jax.ShapeDtypeStruct((2, _N, _HHD), _f32),
        jax.ShapeDtypeStruct((2, _EPAD, 16), _f32),
    ),
    mesh=_MESH,
    compiler_params=pltpu.CompilerParams(use_tc_tiling_on_sc=False),
    scratch_types=[
        pltpu.VMEM_SHARED((_N, _HHD), _f32),
        pltpu.VMEM_SHARED((_N, _HHD), _f32),
        pltpu.VMEM_SHARED((_N, 16), _f32),
        pltpu.VMEM_SHARED((_N, 16), _f32),
        pltpu.VMEM_SHARED((_N, 16), _f32),
        pltpu.VMEM((3, 2, _BS), jnp.int32),
        pltpu.VMEM((3, _BS, _HHD), _f32),
        pltpu.VMEM((3, _BS, 16), _f32),
        pltpu.VMEM((32, _HHD), _f32),
        pltpu.VMEM((_BS, 16), _f32),
        pltpu.VMEM((_BS, 16), _f32),
        pltpu.VMEM((32, 16), _f32),
        pltpu.SemaphoreType.DMA((3,)),
        pltpu.SemaphoreType.DMA((3,)),
        pltpu.SemaphoreType.DMA((3,)),
        pltpu.SemaphoreType.DMA((3,)),
    ],
)
def _sc_diffuse(feat2, eh2, et2, sd4, z64, z16, fout, wout,
                f_sp, acc_sp, eh_sp, et_sp, den_sp,
                sd, rows, wbuf, f0b, ehg, etg, dbuf, semi, semg, sems, semw):
    c = lax.axis_index("c")
    t = lax.axis_index("s")
    node0 = t * _NPT
    lanes = lax.broadcasted_iota(jnp.int32, (16,), 0)
    head_mask = lanes < 4

    pltpu.sync_copy(feat2.at[c, pl.ds(node0, _NPT)], f_sp.at[pl.ds(node0, _NPT)])
    pltpu.sync_copy(eh2.at[c, pl.ds(node0, _NPT)], eh_sp.at[pl.ds(node0, _NPT)])
    pltpu.sync_copy(et2.at[c, pl.ds(node0, _NPT)], et_sp.at[pl.ds(node0, _NPT)])
    for n0, sz in _UCHUNKS:
        pltpu.sync_copy(z64.at[pl.ds(0, sz)],
                        acc_sp.at[pl.ds(node0 + n0, sz)])
    pltpu.sync_copy(z16, den_sp.at[pl.ds(node0, _NPT)])
    plsc.subcore_barrier()

    nb = jnp.where(t == _NT - 1, _NB15, _NB)

    def _bg_eh(b):
        return pltpu.make_async_copy(eh_sp.at[sd.at[b, 0]], ehg, semg.at[b])

    def _bg_et(b):
        return pltpu.make_async_copy(et_sp.at[sd.at[b, 1]], etg, semi.at[b])

    def _bo_den(b):
        return pltpu.make_async_copy(wbuf.at[b], den_sp.at[sd.at[b, 1]],
                                     sems.at[b])

    def _bo_w(j, b):
        return pltpu.make_async_copy(
            wbuf.at[b], wout.at[c, pl.ds(t * _EPT + j * _BS, _BS)],
            semw.at[b])

    def _bbatch(j, b):
        @pl.when(j >= 3)
        def _drain():
            _bo_den(b).wait()
            _bo_w(0, b).wait()

        pltpu.sync_copy(sd4.at[t, j], sd.at[b])
        _bg_eh(b).start()
        _bg_et(b).start()
        _bg_eh(b).wait()
        _bg_et(b).wait()

        def _wrow(r, carry2):
            s = ehg[r, :] + etg[r, :]
            l = jnp.maximum(s, 0.2 * s)
            w = jnp.where(head_mask, jnp.exp(l), 0.0)
            wbuf[b, r, :] = w
            return carry2

        lax.fori_loop(0, _BS, _wrow, 0, unroll=4)
        _bo_den(b).start(add=True)
        _bo_w(j, b).start()

    def _btriple(jj, carry):
        for b in (0, 1, 2):
            _bbatch(3 * jj + b, b)
        return carry

    lax.fori_loop(0, nb // 3, _btriple, 0)
    _bbatch(nb - 2, 0)
    _bbatch(nb - 1, 1)
    _bo_den(2).wait()
    _bo_w(0, 2).wait()
    _bo_den(0).wait()
    _bo_w(0, 0).wait()
    _bo_den(1).wait()
    _bo_w(0, 1).wait()
    plsc.subcore_barrier()

    def _in_sd(j, b):
        return pltpu.make_async_copy(sd4.at[t, j], sd.at[b], semi.at[b])

    def _in_w(j, b):
        return pltpu.make_async_copy(
            wout.at[c, pl.ds(t * _EPT + j * _BS, _BS)], wbuf.at[b],
            semi.at[b])

    def _issue_in(j, b):
        _in_sd(j, b).start()
        _in_w(j, b).start()

    def _wait_in(b):
        _in_sd(0, b).wait()
        _in_w(0, b).wait()

    def _gather(b):
        return pltpu.make_async_copy(f_sp.at[sd.at[b, 0]], rows.at[b],
                                     semg.at[b])

    def _scatter(b):
        return pltpu.make_async_copy(rows.at[b], acc_sp.at[sd.at[b, 1]],
                                     sems.at[b])

    def _scale(b):
        def _srow(r, carry2):
            wv16 = wbuf[b, r, :]
            for k in range(4):
                wv = wv16[k]
                rows[b, r, pl.ds(k * 16, 16)] = (
                    rows[b, r, pl.ds(k * 16, 16)] * wv)
            return carry2

        lax.fori_loop(0, _BS, _srow, 0, unroll=4)

    def _hop_body(_hop, hcarry):
        _issue_in(0, 0)
        _issue_in(1, 1)
        _wait_in(0)
        _gather(0).start()

        def _triple(jj, carry):
            for b in (0, 1, 2):
                j = 3 * jj + b
                b1 = (b + 1) % 3
                b2 = (b + 2) % 3

                @pl.when(j + 2 < nb)
                def _prefetch_in():
                    @pl.when(j >= 1)
                    def _drain():
                        _scatter(b2).wait()

                    _issue_in(j + 2, b2)

                @pl.when(j + 1 < nb)
                def _prefetch_gather():
                    _wait_in(b1)
                    _gather(b1).start()

                _gather(b).wait()
                _scale(b)
                _scatter(b).start(add=True)
            return carry

        lax.fori_loop(0, nb // 3, _triple, 0)

        _wait_in(1)
        _gather(1).start()
        _gather(0).wait()
        _scale(0)
        _scatter(0).start(add=True)
        _gather(1).wait()
        _scale(1)
        _scatter(1).start(add=True)
        _scatter(2).wait()
        _scatter(0).wait()
        _scatter(1).wait()
        plsc.subcore_barrier()

        for n0r, sz in _UCHUNKS:
            n0 = node0 + n0r
            pltpu.sync_copy(acc_sp.at[pl.ds(n0, sz)], rows.at[0, pl.ds(0, sz)])
            pltpu.sync_copy(feat2.at[c, pl.ds(n0, sz)], f0b.at[pl.ds(0, sz)])
            pltpu.sync_copy(den_sp.at[pl.ds(n0, sz)], dbuf.at[pl.ds(0, sz)])

            def _urow(r, carry2):
                d16 = dbuf[r, :]
                inv16 = (1.0 - _ALPHA) / jnp.maximum(d16, 1e-30)
                for k in range(4):
                    inv = inv16[k]
                    acc_v = rows[0, r, pl.ds(k * 16, 16)]
                    f0_v = f0b[r, pl.ds(k * 16, 16)]
                    rows[0, r, pl.ds(k * 16, 16)] = acc_v * inv + _ALPHA * f0_v
                return carry2

            lax.fori_loop(0, sz, _urow, 0)
            pltpu.sync_copy(rows.at[0, pl.ds(0, sz)], f_sp.at[pl.ds(n0, sz)])
            pltpu.sync_copy(z64.at[pl.ds(0, sz)], acc_sp.at[pl.ds(n0, sz)])
        plsc.subcore_barrier()
        return hcarry

    lax.fori_loop(0, _HOP, _hop_body, 0)

    pltpu.sync_copy(f_sp.at[pl.ds(node0, _NPT)], fout.at[c, pl.ds(node0, _NPT)])


def kernel(ent_feat, edge_index, ln1_g, ln1_b, W_ent, attn_h, attn_t,
           ln2_g, ln2_b, w1, b1, w2, b2):
    ahf = attn_h.reshape(1, _D)
    atf = attn_t.reshape(1, _D)
    mh = jnp.repeat(jnp.eye(_H, dtype=_f32), _DH, axis=0)
    g1 = ln1_g.reshape(1, _D)
    b1n = ln1_b.reshape(1, _D)

    h, feat, eh, et = _proj(ent_feat, g1, b1n, W_ent, ahf, atf, mh)

    feat2 = feat.reshape(_N, 2, _HHD).transpose(1, 0, 2)
    eh2 = jnp.pad(eh.reshape(_N, 2, 4).transpose(1, 0, 2),
                  ((0, 0), (0, 0), (0, 12)))
    et2 = jnp.pad(et.reshape(_N, 2, 4).transpose(1, 0, 2),
                  ((0, 0), (0, 0), (0, 12)))
    pad = _EPAD - _E
    sd4 = jnp.pad(edge_index, ((0, 0), (0, pad)))
    sd4 = sd4.reshape(2, _NT, _NB, _BS).transpose(1, 2, 0, 3)

    z64 = jnp.zeros((32, _HHD), _f32)
    z16 = jnp.zeros((_NPT, 16), _f32)
    fout, _ = _sc_diffuse(feat2, eh2, et2, sd4, z64, z16)
    f = fout.transpose(1, 0, 2).reshape(_N, _D)

    return _ffn(f, h, ln2_g.reshape(1, _D), ln2_b.reshape(1, _D),
                w1, b1.reshape(1, _DFF), w2, b2.reshape(1, _D))


# speedup vs baseline: 1.1811x; 1.1811x over previous
import functools

import jax
import jax.numpy as jnp
from jax import lax
from jax.experimental import pallas as pl
from jax.experimental.pallas import tpu as pltpu
from jax.experimental.pallas import tpu_sc as plsc

_N = 10000
_E = 320000
_D = 128
_H = 8
_DH = 16
_HOP = 5
_ALPHA = 0.15
_DFF = 512

_NT = 16
_BS = 64
_NB = 314
_EPT = _NB * _BS
_EPAD = _NT * _EPT
_NB15 = 290
_NPT = _N // _NT
_UCHUNKS = tuple((q * 32, 32) for q in range(19)) + ((608, 17),)
_HHD = 64

_f32 = jnp.float32


def _proj_body(x_ref, g_ref, b_ref, w_ref, ahf_ref, atf_ref, mh_ref,
               h_ref, feat_ref, eh_ref, et_ref):
    x = x_ref[...]
    m = jnp.mean(x, axis=1, keepdims=True)
    xc = x - m
    var = jnp.mean(xc * xc, axis=1, keepdims=True)
    h = xc * lax.rsqrt(var + 1e-5) * g_ref[...] + b_ref[...]
    feat = lax.dot_general(h, w_ref[...], (((1,), (1,)), ((), ())),
                           preferred_element_type=_f32)
    eh = lax.dot_general(feat * ahf_ref[...], mh_ref[...],
                         (((1,), (0,)), ((), ())), preferred_element_type=_f32)
    et = lax.dot_general(feat * atf_ref[...], mh_ref[...],
                         (((1,), (0,)), ((), ())), preferred_element_type=_f32)
    h_ref[...] = h
    feat_ref[...] = feat
    eh_ref[...] = eh
    et_ref[...] = et


_PROJ_NB = 400


def _proj(x, g, b, w, ahf, atf, mh):
    grid = _N // _PROJ_NB
    full = lambda i: (0, 0)
    blk = lambda i: (i, 0)
    return pl.pallas_call(
        _proj_body,
        grid=(grid,),
        in_specs=[
            pl.BlockSpec((_PROJ_NB, _D), blk),
            pl.BlockSpec((1, _D), full),
            pl.BlockSpec((1, _D), full),
            pl.BlockSpec((_D, _D), full),
            pl.BlockSpec((1, _D), full),
            pl.BlockSpec((1, _D), full),
            pl.BlockSpec((_D, _H), full),
        ],
        out_specs=[
            pl.BlockSpec((_PROJ_NB, _D), blk),
            pl.BlockSpec((_PROJ_NB, _D), blk),
            pl.BlockSpec((_PROJ_NB, _H), blk),
            pl.BlockSpec((_PROJ_NB, _H), blk),
        ],
        out_shape=[
            jax.ShapeDtypeStruct((_N, _D), _f32),
            jax.ShapeDtypeStruct((_N, _D), _f32),
            jax.ShapeDtypeStruct((_N, _H), _f32),
            jax.ShapeDtypeStruct((_N, _H), _f32),
        ],
    )(x, g, b, w, ahf, atf, mh)


def _ffn_body(f_ref, h_ref, g_ref, b_ref, w1_ref, b1_ref, w2_ref, b2_ref,
              o_ref):
    rst = f_ref[...] + h_ref[...]
    m = jnp.mean(rst, axis=1, keepdims=True)
    xc = rst - m
    var = jnp.mean(xc * xc, axis=1, keepdims=True)
    h2 = xc * lax.rsqrt(var + 1e-5) * g_ref[...] + b_ref[...]
    t = lax.dot_general(h2, w1_ref[...], (((1,), (1,)), ((), ())),
                        preferred_element_type=_f32) + b1_ref[...]
    t = jnp.maximum(t, 0.0)
    o_ref[...] = lax.dot_general(t, w2_ref[...], (((1,), (1,)), ((), ())),
                                 preferred_element_type=_f32) + b2_ref[...] + rst


def _ffn(f, h, g, b, w1, b1, w2, b2):
    grid = _N // _PROJ_NB
    full = lambda i: (0, 0)
    blk = lambda i: (i, 0)
    return pl.pallas_call(
        _ffn_body,
        grid=(grid,),
        in_specs=[
            pl.BlockSpec((_PROJ_NB, _D), blk),
            pl.BlockSpec((_PROJ_NB, _D), blk),
            pl.BlockSpec((1, _D), full),
            pl.BlockSpec((1, _D), full),
            pl.BlockSpec((_DFF, _D), full),
            pl.BlockSpec((1, _DFF), full),
            pl.BlockSpec((_D, _DFF), full),
            pl.BlockSpec((1, _D), full),
        ],
        out_specs=pl.BlockSpec((_PROJ_NB, _D), blk),
        out_shape=jax.ShapeDtypeStruct((_N, _D), _f32),
    )(f, h, g, b, w1, b1, w2, b2)


_MESH = plsc.VectorSubcoreMesh(core_axis_name="c", subcore_axis_name="s")


@functools.partial(
    pl.kernel,
    out_type=(
        jax.ShapeDtypeStruct((2, _N, _HHD), _f32),
        jax.ShapeDtypeStruct((2, _EPAD, 16), _f32),
    ),
    mesh=_MESH,
    compiler_params=pltpu.CompilerParams(use_tc_tiling_on_sc=False),
    scratch_types=[
        pltpu.VMEM_SHARED((_N, _HHD), _f32),
        pltpu.VMEM_SHARED((_N, _HHD), _f32),
        pltpu.VMEM_SHARED((_N, 16), _f32),
        pltpu.VMEM_SHARED((_N, 16), _f32),
        pltpu.VMEM_SHARED((_N, 16), _f32),
        pltpu.VMEM((3, 2, _BS), jnp.int32),
        pltpu.VMEM((3, _BS, _HHD), _f32),
        pltpu.VMEM((3, _BS, 16), _f32),
        pltpu.VMEM((32, _HHD), _f32),
        pltpu.VMEM((_BS, 16), _f32),
        pltpu.VMEM((_BS, 16), _f32),
        pltpu.VMEM((32, 16), _f32),
        pltpu.SemaphoreType.DMA((3,)),
        pltpu.SemaphoreType.DMA((3,)),
        pltpu.SemaphoreType.DMA((3,)),
        pltpu.SemaphoreType.DMA((3,)),
        pltpu.SemaphoreType.DMA((3,)),
        pltpu.SemaphoreType.DMA((3,)),
        pltpu.SemaphoreType.DMA((3,)),
    ],
)
def _sc_diffuse(feat2, eh2, et2, sd4, z64, z16, fout, wout,
                f_sp, acc_sp, eh_sp, et_sp, den_sp,
                sd, rows, wbuf, f0b, ehg, etg, dbuf, semi, semg, sems, semw,
                semr2, semr3, semz):
    c = lax.axis_index("c")
    t = lax.axis_index("s")
    node0 = t * _NPT
    lanes = lax.broadcasted_iota(jnp.int32, (16,), 0)
    head_mask = lanes < 4

    pltpu.sync_copy(feat2.at[c, pl.ds(node0, _NPT)], f_sp.at[pl.ds(node0, _NPT)])
    pltpu.sync_copy(eh2.at[c, pl.ds(node0, _NPT)], eh_sp.at[pl.ds(node0, _NPT)])
    pltpu.sync_copy(et2.at[c, pl.ds(node0, _NPT)], et_sp.at[pl.ds(node0, _NPT)])
    for n0, sz in _UCHUNKS:
        pltpu.sync_copy(z64.at[pl.ds(0, sz)],
                        acc_sp.at[pl.ds(node0 + n0, sz)])
    pltpu.sync_copy(z16, den_sp.at[pl.ds(node0, _NPT)])
    plsc.subcore_barrier()

    nb = jnp.where(t == _NT - 1, _NB15, _NB)

    def _bg_eh(b):
        return pltpu.make_async_copy(eh_sp.at[sd.at[b, 0]], ehg, semg.at[b])

    def _bg_et(b):
        return pltpu.make_async_copy(et_sp.at[sd.at[b, 1]], etg, semi.at[b])

    def _bo_den(b):
        return pltpu.make_async_copy(wbuf.at[b], den_sp.at[sd.at[b, 1]],
                                     sems.at[b])

    def _bo_w(j, b):
        return pltpu.make_async_copy(
            wbuf.at[b], wout.at[c, pl.ds(t * _EPT + j * _BS, _BS)],
            semw.at[b])

    def _bbatch(j, b):
        @pl.when(j >= 3)
        def _drain():
            _bo_den(b).wait()
            _bo_w(0, b).wait()

        pltpu.sync_copy(sd4.at[t, j], sd.at[b])
        _bg_eh(b).start()
        _bg_et(b).start()
        _bg_eh(b).wait()
        _bg_et(b).wait()

        def _wrow(r, carry2):
            s = ehg[r, :] + etg[r, :]
            l = jnp.maximum(s, 0.2 * s)
            w = jnp.where(head_mask, jnp.exp(l), 0.0)
            wbuf[b, r, :] = w
            return carry2

        lax.fori_loop(0, _BS, _wrow, 0)
        _bo_den(b).start(add=True)
        _bo_w(j, b).start()

    def _btriple(jj, carry):
        for b in (0, 1, 2):
            _bbatch(3 * jj + b, b)
        return carry

    lax.fori_loop(0, nb // 3, _btriple, 0)
    _bbatch(nb - 2, 0)
    _bbatch(nb - 1, 1)
    _bo_den(2).wait()
    _bo_w(0, 2).wait()
    _bo_den(0).wait()
    _bo_w(0, 0).wait()
    _bo_den(1).wait()
    _bo_w(0, 1).wait()
    plsc.subcore_barrier()

    def _in_sd(j, b):
        return pltpu.make_async_copy(sd4.at[t, j], sd.at[b], semi.at[b])

    def _in_w(j, b):
        return pltpu.make_async_copy(
            wout.at[c, pl.ds(t * _EPT + j * _BS, _BS)], wbuf.at[b],
            semi.at[b])

    def _issue_in(j, b):
        _in_sd(j, b).start()
        _in_w(j, b).start()

    def _wait_in(b):
        _in_sd(0, b).wait()
        _in_w(0, b).wait()

    def _gather(b):
        return pltpu.make_async_copy(f_sp.at[sd.at[b, 0]], rows.at[b],
                                     semg.at[b])

    def _scatter(b):
        return pltpu.make_async_copy(rows.at[b], acc_sp.at[sd.at[b, 1]],
                                     sems.at[b])

    def _scale(b):
        def _srow(r, carry2):
            wv16 = wbuf[b, r, :]
            for k in range(4):
                wv = wv16[k]
                rows[b, r, pl.ds(k * 16, 16)] = (
                    rows[b, r, pl.ds(k * 16, 16)] * wv)
            return carry2

        lax.fori_loop(0, _BS, _srow, 0)

    def _hop_body(_hop, hcarry):
        _issue_in(0, 0)
        _issue_in(1, 1)
        _wait_in(0)
        _gather(0).start()

        def _triple(jj, carry):
            for b in (0, 1, 2):
                j = 3 * jj + b
                b1 = (b + 1) % 3
                b2 = (b + 2) % 3

                @pl.when(j + 2 < nb)
                def _prefetch_in():
                    @pl.when(j >= 1)
                    def _drain():
                        _scatter(b2).wait()

                    _issue_in(j + 2, b2)

                @pl.when(j + 1 < nb)
                def _prefetch_gather():
                    _wait_in(b1)
                    _gather(b1).start()

                _gather(b).wait()
                _scale(b)
                _scatter(b).start(add=True)
            return carry

        lax.fori_loop(0, nb // 3, _triple, 0)

        _wait_in(1)
        _gather(1).start()
        _gather(0).wait()
        _scale(0)
        _scatter(0).start(add=True)
        _gather(1).wait()
        _scale(1)
        _scatter(1).start(add=True)
        _scatter(2).wait()
        _scatter(0).wait()
        _scatter(1).wait()
        plsc.subcore_barrier()

        def _u_reads(q, b):
            n0r, sz = _UCHUNKS[q]
            n0 = node0 + n0r
            pltpu.async_copy(acc_sp.at[pl.ds(n0, sz)],
                             rows.at[b, pl.ds(0, sz)], semi.at[b])
            pltpu.async_copy(feat2.at[c, pl.ds(n0, sz)],
                             rows.at[b, pl.ds(32, sz)], semr2.at[b])
            pltpu.async_copy(den_sp.at[pl.ds(n0, sz)],
                             wbuf.at[b, pl.ds(0, sz)], semr3.at[b])

        def _u_wait_reads(q, b):
            n0r, sz = _UCHUNKS[q]
            n0 = node0 + n0r
            pltpu.make_async_copy(acc_sp.at[pl.ds(n0, sz)],
                                  rows.at[b, pl.ds(0, sz)], semi.at[b]).wait()
            pltpu.make_async_copy(feat2.at[c, pl.ds(n0, sz)],
                                  rows.at[b, pl.ds(32, sz)], semr2.at[b]).wait()
            pltpu.make_async_copy(den_sp.at[pl.ds(n0, sz)],
                                  wbuf.at[b, pl.ds(0, sz)], semr3.at[b]).wait()

        def _u_writes(q, b, start):
            n0r, sz = _UCHUNKS[q]
            n0 = node0 + n0r
            fw = pltpu.make_async_copy(rows.at[b, pl.ds(0, sz)],
                                       f_sp.at[pl.ds(n0, sz)], semw.at[b])
            zw = pltpu.make_async_copy(z64.at[pl.ds(0, sz)],
                                       acc_sp.at[pl.ds(n0, sz)], semz.at[b])
            if start:
                fw.start()
                zw.start()
            else:
                fw.wait()
                zw.wait()

        nchunks = len(_UCHUNKS)
        _u_reads(0, 0)
        for q in range(nchunks):
            b = q % 3
            if q + 1 < nchunks:
                bn = (q + 1) % 3
                if q >= 2:
                    _u_writes(q - 2, bn, False)
                _u_reads(q + 1, bn)
            _u_wait_reads(q, b)
            sz = _UCHUNKS[q][1]

            def _urow(r, carry2):
                d16 = wbuf[b, r, :]
                inv16 = (1.0 - _ALPHA) / jnp.maximum(d16, 1e-30)
                for k in range(4):
                    inv = inv16[k]
                    acc_v = rows[b, r, pl.ds(k * 16, 16)]
                    f0_v = rows[b, 32 + r, pl.ds(k * 16, 16)]
                    rows[b, r, pl.ds(k * 16, 16)] = acc_v * inv + _ALPHA * f0_v
                return carry2

            lax.fori_loop(0, sz, _urow, 0)
            _u_writes(q, b, True)
        for q in range(nchunks - 3, nchunks):
            _u_writes(q, q % 3, False)
        plsc.subcore_barrier()
        return hcarry

    lax.fori_loop(0, _HOP, _hop_body, 0)

    pltpu.sync_copy(f_sp.at[pl.ds(node0, _NPT)], fout.at[c, pl.ds(node0, _NPT)])


def kernel(ent_feat, edge_index, ln1_g, ln1_b, W_ent, attn_h, attn_t,
           ln2_g, ln2_b, w1, b1, w2, b2):
    ahf = attn_h.reshape(1, _D)
    atf = attn_t.reshape(1, _D)
    mh = jnp.repeat(jnp.eye(_H, dtype=_f32), _DH, axis=0)
    g1 = ln1_g.reshape(1, _D)
    b1n = ln1_b.reshape(1, _D)

    h, feat, eh, et = _proj(ent_feat, g1, b1n, W_ent, ahf, atf, mh)

    feat2 = feat.reshape(_N, 2, _HHD).transpose(1, 0, 2)
    eh2 = jnp.pad(eh.reshape(_N, 2, 4).transpose(1, 0, 2),
                  ((0, 0), (0, 0), (0, 12)))
    et2 = jnp.pad(et.reshape(_N, 2, 4).transpose(1, 0, 2),
                  ((0, 0), (0, 0), (0, 12)))
    pad = _EPAD - _E
    sd4 = jnp.pad(edge_index, ((0, 0), (0, pad)))
    sd4 = sd4.reshape(2, _NT, _NB, _BS).transpose(1, 2, 0, 3)

    z64 = jnp.zeros((32, _HHD), _f32)
    z16 = jnp.zeros((_NPT, 16), _f32)
    fout, _ = _sc_diffuse(feat2, eh2, et2, sd4, z64, z16)
    f = fout.transpose(1, 0, 2).reshape(_N, _D)

    return _ffn(f, h, ln2_g.reshape(1, _D), ln2_b.reshape(1, _D),
                w1, b1.reshape(1, _DFF), w2, b2.reshape(1, _D))


# speedup vs baseline: 1.3253x; 1.1221x over previous
import functools

import jax
import jax.numpy as jnp
from jax import lax
from jax.experimental import pallas as pl
from jax.experimental.pallas import tpu as pltpu
from jax.experimental.pallas import tpu_sc as plsc

_N = 10000
_E = 320000
_D = 128
_H = 8
_DH = 16
_HOP = 5
_ALPHA = 0.15
_DFF = 512

_NT = 16
_BS = 64
_NB = 314
_EPT = _NB * _BS
_EPAD = _NT * _EPT
_NB15 = 290
_NPT = _N // _NT
_UCHUNKS = tuple((q * 32, 32) for q in range(19)) + ((608, 17),)
_HHD = 64

_f32 = jnp.float32


def _proj_body(x_ref, g_ref, b_ref, w_ref, ahf_ref, atf_ref, mh_ref,
               h_ref, feat_ref, eh_ref, et_ref):
    x = x_ref[...]
    m = jnp.mean(x, axis=1, keepdims=True)
    xc = x - m
    var = jnp.mean(xc * xc, axis=1, keepdims=True)
    h = xc * lax.rsqrt(var + 1e-5) * g_ref[...] + b_ref[...]
    feat = lax.dot_general(h, w_ref[...], (((1,), (1,)), ((), ())),
                           preferred_element_type=_f32)
    eh = lax.dot_general(feat * ahf_ref[...], mh_ref[...],
                         (((1,), (0,)), ((), ())), preferred_element_type=_f32)
    et = lax.dot_general(feat * atf_ref[...], mh_ref[...],
                         (((1,), (0,)), ((), ())), preferred_element_type=_f32)
    h_ref[...] = h
    feat_ref[...] = feat
    eh_ref[...] = eh
    et_ref[...] = et


_PROJ_NB = 400


def _proj(x, g, b, w, ahf, atf, mh):
    grid = _N // _PROJ_NB
    full = lambda i: (0, 0)
    blk = lambda i: (i, 0)
    return pl.pallas_call(
        _proj_body,
        grid=(grid,),
        in_specs=[
            pl.BlockSpec((_PROJ_NB, _D), blk),
            pl.BlockSpec((1, _D), full),
            pl.BlockSpec((1, _D), full),
            pl.BlockSpec((_D, _D), full),
            pl.BlockSpec((1, _D), full),
            pl.BlockSpec((1, _D), full),
            pl.BlockSpec((_D, _H), full),
        ],
        out_specs=[
            pl.BlockSpec((_PROJ_NB, _D), blk),
            pl.BlockSpec((_PROJ_NB, _D), blk),
            pl.BlockSpec((_PROJ_NB, _H), blk),
            pl.BlockSpec((_PROJ_NB, _H), blk),
        ],
        out_shape=[
            jax.ShapeDtypeStruct((_N, _D), _f32),
            jax.ShapeDtypeStruct((_N, _D), _f32),
            jax.ShapeDtypeStruct((_N, _H), _f32),
            jax.ShapeDtypeStruct((_N, _H), _f32),
        ],
    )(x, g, b, w, ahf, atf, mh)


def _ffn_body(f_ref, h_ref, g_ref, b_ref, w1_ref, b1_ref, w2_ref, b2_ref,
              o_ref):
    rst = f_ref[...] + h_ref[...]
    m = jnp.mean(rst, axis=1, keepdims=True)
    xc = rst - m
    var = jnp.mean(xc * xc, axis=1, keepdims=True)
    h2 = xc * lax.rsqrt(var + 1e-5) * g_ref[...] + b_ref[...]
    t = lax.dot_general(h2, w1_ref[...], (((1,), (1,)), ((), ())),
                        preferred_element_type=_f32) + b1_ref[...]
    t = jnp.maximum(t, 0.0)
    o_ref[...] = lax.dot_general(t, w2_ref[...], (((1,), (1,)), ((), ())),
                                 preferred_element_type=_f32) + b2_ref[...] + rst


def _ffn(f, h, g, b, w1, b1, w2, b2):
    grid = _N // _PROJ_NB
    full = lambda i: (0, 0)
    blk = lambda i: (i, 0)
    return pl.pallas_call(
        _ffn_body,
        grid=(grid,),
        in_specs=[
            pl.BlockSpec((_PROJ_NB, _D), blk),
            pl.BlockSpec((_PROJ_NB, _D), blk),
            pl.BlockSpec((1, _D), full),
            pl.BlockSpec((1, _D), full),
            pl.BlockSpec((_DFF, _D), full),
            pl.BlockSpec((1, _DFF), full),
            pl.BlockSpec((_D, _DFF), full),
            pl.BlockSpec((1, _D), full),
        ],
        out_specs=pl.BlockSpec((_PROJ_NB, _D), blk),
        out_shape=jax.ShapeDtypeStruct((_N, _D), _f32),
    )(f, h, g, b, w1, b1, w2, b2)


_MESH = plsc.VectorSubcoreMesh(core_axis_name="c", subcore_axis_name="s")


@functools.partial(
    pl.kernel,
    out_type=(
        jax.ShapeDtypeStruct((2, _N, _HHD), _f32),
        jax.ShapeDtypeStruct((2, _EPAD, 16), _f32),
    ),
    mesh=_MESH,
    compiler_params=pltpu.CompilerParams(use_tc_tiling_on_sc=False),
    scratch_types=[
        pltpu.VMEM_SHARED((_N, _HHD), _f32),
        pltpu.VMEM_SHARED((_N, _HHD), _f32),
        pltpu.VMEM_SHARED((_N, 16), _f32),
        pltpu.VMEM_SHARED((_N, 16), _f32),
        pltpu.VMEM_SHARED((_N, 16), _f32),
        pltpu.VMEM((6, 2, _BS), jnp.int32),
        pltpu.VMEM((3, _BS, _HHD), _f32),
        pltpu.VMEM((3, _BS, 16), _f32),
        pltpu.VMEM((2, _BS, 16), _f32),
        pltpu.VMEM((2, _BS, 16), _f32),
        pltpu.SemaphoreType.DMA((3,)),
        pltpu.SemaphoreType.DMA((3,)),
        pltpu.SemaphoreType.DMA((3,)),
        pltpu.SemaphoreType.DMA((3,)),
        pltpu.SemaphoreType.DMA((3,)),
        pltpu.SemaphoreType.DMA((3,)),
        pltpu.SemaphoreType.DMA((3,)),
        pltpu.SemaphoreType.DMA((6,)),
    ],
)
def _sc_diffuse(feat2, eh2, et2, sd4, z64, z16, fout, wout,
                f_sp, acc_sp, eh_sp, et_sp, den_sp,
                sd, rows, wbuf, ehg, etg, semi, semg, sems, semw,
                semr2, semr3, semz, semb):
    c = lax.axis_index("c")
    t = lax.axis_index("s")
    node0 = t * _NPT
    lanes = lax.broadcasted_iota(jnp.int32, (16,), 0)
    head_mask = lanes < 4

    pltpu.sync_copy(feat2.at[c, pl.ds(node0, _NPT)], f_sp.at[pl.ds(node0, _NPT)])
    pltpu.sync_copy(eh2.at[c, pl.ds(node0, _NPT)], eh_sp.at[pl.ds(node0, _NPT)])
    pltpu.sync_copy(et2.at[c, pl.ds(node0, _NPT)], et_sp.at[pl.ds(node0, _NPT)])
    for n0, sz in _UCHUNKS:
        pltpu.sync_copy(z64.at[pl.ds(0, sz)],
                        acc_sp.at[pl.ds(node0 + n0, sz)])
    pltpu.sync_copy(z16, den_sp.at[pl.ds(node0, _NPT)])
    plsc.subcore_barrier()

    nb = jnp.where(t == _NT - 1, _NB15, _NB)

    def _b_in(j, s6):
        return pltpu.make_async_copy(sd4.at[t, j], sd.at[s6], semb.at[s6])

    def _bg_eh(g2, s6):
        return pltpu.make_async_copy(eh_sp.at[sd.at[s6, 0]], ehg.at[g2],
                                     semg.at[g2])

    def _bg_et(g2, s6):
        return pltpu.make_async_copy(et_sp.at[sd.at[s6, 1]], etg.at[g2],
                                     semi.at[g2])

    def _bo_den(w3, s6):
        return pltpu.make_async_copy(wbuf.at[w3], den_sp.at[sd.at[s6, 1]],
                                     sems.at[w3])

    def _bo_w(j, w3):
        return pltpu.make_async_copy(
            wbuf.at[w3], wout.at[c, pl.ds(t * _EPT + j * _BS, _BS)],
            semw.at[w3])

    def _bbatch(j, s6, g2, w3):
        @pl.when(j >= 3)
        def _drain():
            _bo_den(w3, (s6 + 3) % 6).wait()
            _bo_w(0, w3).wait()

        @pl.when(j + 3 < nb)
        def _next_in():
            _b_in(j + 3, (s6 + 3) % 6).start()

        @pl.when(j + 1 < nb)
        def _next_gather():
            _b_in(0, (s6 + 1) % 6).wait()
            _bg_eh((g2 + 1) % 2, (s6 + 1) % 6).start()
            _bg_et((g2 + 1) % 2, (s6 + 1) % 6).start()

        _bg_eh(g2, s6).wait()
        _bg_et(g2, s6).wait()

        def _wrow(r, carry2):
            s = ehg[g2, r, :] + etg[g2, r, :]
            l = jnp.maximum(s, 0.2 * s)
            w = jnp.where(head_mask, jnp.exp(l), 0.0)
            wbuf[w3, r, :] = w
            return carry2

        lax.fori_loop(0, _BS, _wrow, 0)
        _bo_den(w3, s6).start(add=True)
        _bo_w(j, w3).start()

    _b_in(0, 0).start()
    _b_in(1, 1).start()
    _b_in(2, 2).start()
    _b_in(0, 0).wait()
    _bg_eh(0, 0).start()
    _bg_et(0, 0).start()

    def _bsix(jj, carry):
        for b in range(6):
            _bbatch(6 * jj + b, b, b % 2, b % 3)
        return carry

    lax.fori_loop(0, nb // 6, _bsix, 0)
    _bbatch(nb - 2, 0, 0, 0)
    _bbatch(nb - 1, 1, 1, 1)
    _bo_den(2, 2).wait()
    _bo_w(0, 2).wait()
    _bo_den(0, 0).wait()
    _bo_w(0, 0).wait()
    _bo_den(1, 1).wait()
    _bo_w(0, 1).wait()
    plsc.subcore_barrier()

    def _in_sd(j, b):
        return pltpu.make_async_copy(sd4.at[t, j], sd.at[b], semi.at[b])

    def _in_w(j, b):
        return pltpu.make_async_copy(
            wout.at[c, pl.ds(t * _EPT + j * _BS, _BS)], wbuf.at[b],
            semi.at[b])

    def _issue_in(j, b):
        _in_sd(j, b).start()
        _in_w(j, b).start()

    def _wait_in(b):
        _in_sd(0, b).wait()
        _in_w(0, b).wait()

    def _gather(b):
        return pltpu.make_async_copy(f_sp.at[sd.at[b, 0]], rows.at[b],
                                     semg.at[b])

    def _scatter(b):
        return pltpu.make_async_copy(rows.at[b], acc_sp.at[sd.at[b, 1]],
                                     sems.at[b])

    def _scale(b):
        def _srow(r, carry2):
            wv16 = wbuf[b, r, :]
            for k in range(4):
                wv = wv16[k]
                rows[b, r, pl.ds(k * 16, 16)] = (
                    rows[b, r, pl.ds(k * 16, 16)] * wv)
            return carry2

        lax.fori_loop(0, _BS, _srow, 0)

    def _hop_body(_hop, hcarry):
        _issue_in(0, 0)
        _issue_in(1, 1)
        _wait_in(0)
        _gather(0).start()

        def _triple(jj, carry):
            for b in (0, 1, 2):
                j = 3 * jj + b
                b1 = (b + 1) % 3
                b2 = (b + 2) % 3

                @pl.when(j + 2 < nb)
                def _prefetch_in():
                    @pl.when(j >= 1)
                    def _drain():
                        _scatter(b2).wait()

                    _issue_in(j + 2, b2)

                @pl.when(j + 1 < nb)
                def _prefetch_gather():
                    _wait_in(b1)
                    _gather(b1).start()

                _gather(b).wait()
                _scale(b)
                _scatter(b).start(add=True)
            return carry

        lax.fori_loop(0, nb // 3, _triple, 0)

        _wait_in(1)
        _gather(1).start()
        _gather(0).wait()
        _scale(0)
        _scatter(0).start(add=True)
        _gather(1).wait()
        _scale(1)
        _scatter(1).start(add=True)
        _scatter(2).wait()
        _scatter(0).wait()
        _scatter(1).wait()
        plsc.subcore_barrier()

        def _u_reads(q, b):
            n0r, sz = _UCHUNKS[q]
            n0 = node0 + n0r
            pltpu.async_copy(acc_sp.at[pl.ds(n0, sz)],
                             rows.at[b, pl.ds(0, sz)], semi.at[b])
            pltpu.async_copy(feat2.at[c, pl.ds(n0, sz)],
                             rows.at[b, pl.ds(32, sz)], semr2.at[b])
            pltpu.async_copy(den_sp.at[pl.ds(n0, sz)],
                             wbuf.at[b, pl.ds(0, sz)], semr3.at[b])

        def _u_wait_reads(q, b):
            n0r, sz = _UCHUNKS[q]
            n0 = node0 + n0r
            pltpu.make_async_copy(acc_sp.at[pl.ds(n0, sz)],
                                  rows.at[b, pl.ds(0, sz)], semi.at[b]).wait()
            pltpu.make_async_copy(feat2.at[c, pl.ds(n0, sz)],
                                  rows.at[b, pl.ds(32, sz)], semr2.at[b]).wait()
            pltpu.make_async_copy(den_sp.at[pl.ds(n0, sz)],
                                  wbuf.at[b, pl.ds(0, sz)], semr3.at[b]).wait()

        def _u_writes(q, b, start):
            n0r, sz = _UCHUNKS[q]
            n0 = node0 + n0r
            fw = pltpu.make_async_copy(rows.at[b, pl.ds(0, sz)],
                                       f_sp.at[pl.ds(n0, sz)], semw.at[b])
            zw = pltpu.make_async_copy(z64.at[pl.ds(0, sz)],
                                       acc_sp.at[pl.ds(n0, sz)], semz.at[b])
            if start:
                fw.start()
                zw.start()
            else:
                fw.wait()
                zw.wait()

        nchunks = len(_UCHUNKS)
        _u_reads(0, 0)
        for q in range(nchunks):
            b = q % 3
            if q + 1 < nchunks:
                bn = (q + 1) % 3
                if q >= 2:
                    _u_writes(q - 2, bn, False)
                _u_reads(q + 1, bn)
            _u_wait_reads(q, b)
            sz = _UCHUNKS[q][1]

            def _urow(r, carry2):
                d16 = wbuf[b, r, :]
                inv16 = (1.0 - _ALPHA) / jnp.maximum(d16, 1e-30)
                for k in range(4):
                    inv = inv16[k]
                    acc_v = rows[b, r, pl.ds(k * 16, 16)]
                    f0_v = rows[b, 32 + r, pl.ds(k * 16, 16)]
                    rows[b, r, pl.ds(k * 16, 16)] = acc_v * inv + _ALPHA * f0_v
                return carry2

            lax.fori_loop(0, sz, _urow, 0)
            _u_writes(q, b, True)
        for q in range(nchunks - 3, nchunks):
            _u_writes(q, q % 3, False)
        plsc.subcore_barrier()
        return hcarry

    lax.fori_loop(0, _HOP, _hop_body, 0)

    pltpu.sync_copy(f_sp.at[pl.ds(node0, _NPT)], fout.at[c, pl.ds(node0, _NPT)])


def kernel(ent_feat, edge_index, ln1_g, ln1_b, W_ent, attn_h, attn_t,
           ln2_g, ln2_b, w1, b1, w2, b2):
    ahf = attn_h.reshape(1, _D)
    atf = attn_t.reshape(1, _D)
    mh = jnp.repeat(jnp.eye(_H, dtype=_f32), _DH, axis=0)
    g1 = ln1_g.reshape(1, _D)
    b1n = ln1_b.reshape(1, _D)

    h, feat, eh, et = _proj(ent_feat, g1, b1n, W_ent, ahf, atf, mh)

    feat2 = feat.reshape(_N, 2, _HHD).transpose(1, 0, 2)
    eh2 = jnp.pad(eh.reshape(_N, 2, 4).transpose(1, 0, 2),
                  ((0, 0), (0, 0), (0, 12)))
    et2 = jnp.pad(et.reshape(_N, 2, 4).transpose(1, 0, 2),
                  ((0, 0), (0, 0), (0, 12)))
    pad = _EPAD - _E
    sd4 = jnp.pad(edge_index, ((0, 0), (0, pad)))
    sd4 = sd4.reshape(2, _NT, _NB, _BS).transpose(1, 2, 0, 3)

    z64 = jnp.zeros((32, _HHD), _f32)
    z16 = jnp.zeros((_NPT, 16), _f32)
    fout, _ = _sc_diffuse(feat2, eh2, et2, sd4, z64, z16)
    f = fout.transpose(1, 0, 2).reshape(_N, _D)

    return _ffn(f, h, ln2_g.reshape(1, _D), ln2_b.reshape(1, _D),
                w1, b1.reshape(1, _DFF), w2, b2.reshape(1, _D))
